# Initial kernel scaffold; baseline (speedup 1.0000x reference)
import jax
import jax.numpy as jnp
from jax.experimental import pallas as pl


def kernel(token_ids, weights):
    raise NotImplementedError("write your pallas kernel here")


# speedup vs baseline: 1.1115x; 1.1115x over previous
import functools

import jax
import jax.numpy as jnp
from jax import lax
from jax.experimental import pallas as pl
from jax.experimental.pallas import tpu as pltpu
from jax.experimental.pallas import tpu_sc as plsc

_D = 32
_NC = 2
_NS = 16
_NW = _NC * _NS
_B = 16384 * 50
_BPW = _B // _NW
_CHUNK = 3200
_NCHUNK = _BPW // _CHUNK


@functools.partial(
    pl.kernel,
    out_type=jax.ShapeDtypeStruct((_B, _D), jnp.float32),
    mesh=plsc.VectorSubcoreMesh(core_axis_name="c", subcore_axis_name="s"),
    scratch_types=[
        pltpu.VMEM((_CHUNK,), jnp.int32),
        pltpu.VMEM((_CHUNK, _D), jnp.float32),
        pltpu.SemaphoreType.DMA,
    ],
    compiler_params=pltpu.CompilerParams(use_tc_tiling_on_sc=False),
)
def _gather_body(ids_hbm, table_hbm, out_hbm, idx_v, rows_v, sem):
    wid = lax.axis_index("s") * _NC + lax.axis_index("c")
    wbase = wid * _BPW

    def step(c, carry):
        base = wbase + c * _CHUNK
        pltpu.sync_copy(ids_hbm.at[pl.ds(base, _CHUNK)], idx_v)
        pltpu.async_copy(table_hbm.at[idx_v], rows_v, sem).wait()
        pltpu.sync_copy(rows_v, out_hbm.at[pl.ds(base, _CHUNK)])
        return carry

    lax.fori_loop(0, _NCHUNK, step, 0)


@jax.jit
def _run(ids, table):
    return _gather_body(ids, table)


def kernel(token_ids, weights):
    ids = token_ids.reshape(-1).astype(jnp.int32)
    out = _run(ids, weights)
    return out.reshape(token_ids.shape + (weights.shape[1],))


# speedup vs baseline: 1.1131x; 1.0015x over previous
import functools

import jax
import jax.numpy as jnp
from jax import lax
from jax.experimental import pallas as pl
from jax.experimental.pallas import tpu as pltpu
from jax.experimental.pallas import tpu_sc as plsc

_D = 32
_NC = 2
_NS = 16
_NW = _NC * _NS
_B = 16384 * 50
_BPW = _B // _NW
_CHUNK = 640
_NCH = _BPW // _CHUNK
_NBUF = 4


@functools.partial(
    pl.kernel,
    out_type=jax.ShapeDtypeStruct((_B, _D), jnp.float32),
    mesh=plsc.VectorSubcoreMesh(core_axis_name="c", subcore_axis_name="s"),
    scratch_types=[
        [pltpu.VMEM((_CHUNK,), jnp.int32)] * _NBUF,
        [pltpu.VMEM((_CHUNK, _D), jnp.float32)] * _NBUF,
        [pltpu.SemaphoreType.DMA] * _NBUF,
        [pltpu.SemaphoreType.DMA] * _NBUF,
        [pltpu.SemaphoreType.DMA] * _NBUF,
    ],
    compiler_params=pltpu.CompilerParams(use_tc_tiling_on_sc=False),
)
def _gather_body(ids_hbm, table_hbm, out_hbm, idxb, rows, isem, gsem, wsem):
    wid = lax.axis_index("s") * _NC + lax.axis_index("c")
    wbase = wid * _BPW

    def istart(c, b):
        pltpu.async_copy(
            ids_hbm.at[pl.ds(wbase + c * _CHUNK, _CHUNK)], idxb[b], isem[b]
        )

    def iwait(c, b):
        pltpu.make_async_copy(
            ids_hbm.at[pl.ds(wbase + c * _CHUNK, _CHUNK)], idxb[b], isem[b]
        ).wait()

    def gstart(c, b):
        pltpu.async_copy(table_hbm.at[idxb[b]], rows[b], gsem[b])

    def gwait(c, b):
        pltpu.make_async_copy(table_hbm.at[idxb[b]], rows[b], gsem[b]).wait()

    def wstart(c, b):
        pltpu.async_copy(
            rows[b], out_hbm.at[pl.ds(wbase + c * _CHUNK, _CHUNK)], wsem[b]
        )

    def wwait(c, b):
        pltpu.make_async_copy(
            rows[b], out_hbm.at[pl.ds(wbase + c * _CHUNK, _CHUNK)], wsem[b]
        ).wait()

    for b in range(_NBUF - 1):
        istart(b, b)
    for b in range(_NBUF - 1):
        iwait(b, b)
        gstart(b, b)
    istart(_NBUF - 1, _NBUF - 1)

    def group(i, carry):
        for b in range(_NBUF):
            c = i * _NBUF + b
            nb = (b + _NBUF - 1) % _NBUF
            gwait(c, b)
            wstart(c, b)
            n = c + _NBUF - 1

            @pl.when(n < _NCH)
            def _():
                iwait(n, nb)

                @pl.when(c >= 1)
                def _():
                    wwait(c - 1, nb)

                gstart(n, nb)

            m = c + _NBUF

            @pl.when(m < _NCH)
            def _():
                istart(m, b)

        return carry

    lax.fori_loop(0, _NCH // _NBUF, group, 0)

    for k in range(_NBUF):
        c = _NCH - _NBUF + k
        wwait(c, c % _NBUF)


@jax.jit
def _run(ids, table):
    return _gather_body(ids, table)


def kernel(token_ids, weights):
    ids = token_ids.reshape(-1).astype(jnp.int32)
    out = _run(ids, weights)
    return out.reshape(token_ids.shape + (weights.shape[1],))


# speedup vs baseline: 1.7589x; 1.5803x over previous
import functools

import jax
import jax.numpy as jnp
from jax import lax
from jax.experimental import pallas as pl
from jax.experimental.pallas import tpu as pltpu
from jax.experimental.pallas import tpu_sc as plsc

_D = 32
_NC = 2
_NS = 16
_NW = _NC * _NS
_S = 16384
_W = 50
_B = _S * _W
_BPW = _B // _NW
_SPC = 8
_CHUNK = _SPC * _W
_NCH = _BPW // _CHUNK
_NBUF = 2


@functools.partial(
    pl.kernel,
    out_type=jax.ShapeDtypeStruct((_S, _W, _D), jnp.float32),
    mesh=plsc.VectorSubcoreMesh(core_axis_name="c", subcore_axis_name="s"),
    scratch_types=[
        [pltpu.VMEM((_CHUNK,), jnp.int32)] * _NBUF,
        [pltpu.VMEM((_CHUNK, _D), jnp.float32)] * _NBUF,
        [pltpu.SemaphoreType.DMA] * _NBUF,
        [pltpu.SemaphoreType.DMA] * _NBUF,
        [pltpu.SemaphoreType.DMA] * _NBUF,
    ],
    compiler_params=pltpu.CompilerParams(use_tc_tiling_on_sc=False),
)
def _gather_body(ids_hbm, table_hbm, out3_hbm, idxb, rows, isem, gsem, wsem):
    wid = lax.axis_index("s") * _NC + lax.axis_index("c")
    wbase = wid * _BPW
    sbase = wid * (_BPW // _W)

    def istart(c, b):
        pltpu.async_copy(
            ids_hbm.at[pl.ds(wbase + c * _CHUNK, _CHUNK)], idxb[b], isem[b]
        )

    def iwait(c, b):
        pltpu.make_async_copy(
            ids_hbm.at[pl.ds(wbase + c * _CHUNK, _CHUNK)], idxb[b], isem[b]
        ).wait()

    def gstart(c, b):
        pltpu.async_copy(table_hbm.at[idxb[b]], rows[b], gsem[b])

    def gwait(c, b):
        pltpu.make_async_copy(table_hbm.at[idxb[b]], rows[b], gsem[b]).wait()

    def wstart(c, b):
        for k in range(_SPC):
            pltpu.async_copy(
                rows[b].at[pl.ds(k * _W, _W)],
                out3_hbm.at[sbase + c * _SPC + k],
                wsem[b],
            )

    def wwait(c, b):
        for k in range(_SPC):
            pltpu.make_async_copy(
                rows[b].at[pl.ds(k * _W, _W)],
                out3_hbm.at[sbase + c * _SPC + k],
                wsem[b],
            ).wait()

    istart(0, 0)
    iwait(0, 0)
    gstart(0, 0)
    istart(1, 1)

    def group(i, carry):
        for b in range(_NBUF):
            c = i * _NBUF + b
            nb = (b + _NBUF - 1) % _NBUF
            gwait(c, b)
            wstart(c, b)
            n = c + _NBUF - 1

            @pl.when(n < _NCH)
            def _():
                iwait(n, nb)

                @pl.when(c >= 1)
                def _():
                    wwait(c - 1, nb)

                gstart(n, nb)

            m = c + _NBUF

            @pl.when(m < _NCH)
            def _():
                istart(m, b)

        return carry

    lax.fori_loop(0, _NCH // _NBUF, group, 0)

    for k in range(_NBUF):
        c = _NCH - _NBUF + k
        wwait(c, c % _NBUF)


@jax.jit
def _run(ids, table):
    return _gather_body(ids, table)


def kernel(token_ids, weights):
    ids = token_ids.reshape(-1).astype(jnp.int32)
    return _run(ids, weights)
